# baseline (speedup 1.0000x reference)
import jax
import jax.numpy as jnp
from jax.experimental import pallas as pl
from jax.experimental.pallas import tpu as pltpu

H, W = 2048, 4096
NLAB = 3
NCH = 3 + NLAB
HH = H // 2
BINS = 256

BR_A = 128
NJ_A = HH // BR_A
TPB = (BR_A // 16) * (W // 128)
BR_C = 128
NJ_C = HH // BR_C


def _hist_kernel(img_ref, hist_ref, vals, acc):
    j = pl.program_id(2)
    one = jnp.ones((), jnp.bfloat16)
    zero = jnp.zeros((), jnp.bfloat16)

    @pl.when(j == 0)
    def _():
        hist_ref[0, 0] = jnp.zeros((BINS, 128), jnp.float32)

    vals[...] = jnp.clip(jnp.floor(img_ref[0]), 0.0, 255.0).astype(jnp.bfloat16)
    acc[...] = jnp.zeros((BINS, 16, 128), jnp.bfloat16)

    def tile_body(t, carry):
        g = jax.lax.rem(t, BR_A // 16)
        cs = pl.multiple_of(jax.lax.div(t, BR_A // 16) * 128, 128)
        tl = vals[pl.ds(g * 16, 16), pl.ds(cs, 128)]
        for b in range(BINS):
            acc[b] = acc[b] + jnp.where(tl == jnp.bfloat16(b), one, zero)
        return carry

    jax.lax.fori_loop(0, TPB, tile_body, 0)

    for b in range(BINS):
        a32 = acc[b].astype(jnp.float32)
        r = jnp.sum(a32, axis=0, keepdims=True)
        hist_ref[0, 0, b:b + 1, :] = hist_ref[0, 0, b:b + 1, :] + r


def _fdiv(a, d):
    q = jnp.floor(a / d)
    r = a - q * d
    for _ in range(3):
        over = r >= d
        q = q + jnp.where(over, 1.0, 0.0)
        r = r - jnp.where(over, d, 0.0)
        under = r < 0.0
        q = q - jnp.where(under, 1.0, 0.0)
        r = r + jnp.where(under, d, 0.0)
    return q


def _shift_rows_down(x, k):
    return jnp.pad(x, ((k, 0), (0, 0)))[:BINS, :]


def _lut_kernel(hist_ref, lut_ref):
    idx = jax.lax.broadcasted_iota(jnp.int32, (BINS, 1), 0).astype(jnp.float32)

    for c in range(3):
        hl = hist_ref[0, c] + hist_ref[1, c]
        h = jnp.sum(hl, axis=-1, keepdims=True)
        cum = h
        for k in (1, 2, 4, 8, 16, 32, 64, 128):
            cum = cum + _shift_rows_down(cum, k)
        total = jnp.sum(h, keepdims=True)
        masked = jnp.where(h > 0.0, idx, -1.0)
        last_nz = jnp.max(masked, keepdims=True)
        h_last = jnp.sum(jnp.where(idx == last_nz, h, 0.0), keepdims=True)
        step = _fdiv(total - h_last, jnp.full((1, 1), 255.0))
        d = jnp.maximum(step, 1.0)
        a = cum + jnp.floor(step * 0.5)
        q = _fdiv(a, d)
        lut = jnp.clip(_shift_rows_down(q, 1), 0.0, 255.0)
        lut = jnp.where(step == 0.0, idx, lut)
        lut_ref[c] = lut


def _remap_kernel(img_ref, lut_ref, out_ref):
    c = pl.program_id(1)

    @pl.when(c < 3)
    def _():
        v = img_ref[0]
        idx = jnp.clip(v, 0.0, 255.0).astype(jnp.int32)
        idxm = jnp.bitwise_and(idx, 127)
        lo_rows = jnp.broadcast_to(lut_ref[0, 0:1, :], (BR_C, 128))
        hi_rows = jnp.broadcast_to(lut_ref[0, 1:2, :], (BR_C, 128))
        g_lo = jnp.take_along_axis(lo_rows, idxm, axis=1)
        g_hi = jnp.take_along_axis(hi_rows, idxm, axis=1)
        out_ref[0] = jnp.where(idx >= 128, g_hi, g_lo)

    @pl.when(c >= 3)
    def _():
        out_ref[0] = img_ref[0]


def kernel(image):
    part = pl.pallas_call(
        _hist_kernel,
        grid=(2, 3, NJ_A),
        in_specs=[pl.BlockSpec((1, BR_A, W),
                               lambda p, c, j: (c, p * NJ_A + j, 0))],
        out_specs=pl.BlockSpec((1, 1, BINS, 128), lambda p, c, j: (p, c, 0, 0)),
        out_shape=jax.ShapeDtypeStruct((2, 3, BINS, 128), jnp.float32),
        scratch_shapes=[
            pltpu.VMEM((BR_A, W), jnp.bfloat16),
            pltpu.VMEM((BINS, 16, 128), jnp.bfloat16),
        ],
        compiler_params=pltpu.CompilerParams(
            dimension_semantics=("parallel", "arbitrary", "arbitrary")),
    )(image)

    lut3d = pl.pallas_call(
        _lut_kernel,
        grid=(1,),
        in_specs=[pl.BlockSpec((2, 3, BINS, 128), lambda i: (0, 0, 0, 0))],
        out_specs=pl.BlockSpec((3, BINS, 1), lambda i: (0, 0, 0)),
        out_shape=jax.ShapeDtypeStruct((3, BINS, 1), jnp.float32),
        compiler_params=pltpu.CompilerParams(
            dimension_semantics=("arbitrary",)),
    )(part)

    lut = lut3d.reshape(3, 2, 128)

    out = pl.pallas_call(
        _remap_kernel,
        grid=(2, NCH, NJ_C),
        in_specs=[
            pl.BlockSpec((1, BR_C, W), lambda p, c, j: (c, p * NJ_C + j, 0)),
            pl.BlockSpec((1, 2, 128),
                         lambda p, c, j: (jnp.minimum(c, 2), 0, 0)),
        ],
        out_specs=pl.BlockSpec((1, BR_C, W),
                               lambda p, c, j: (c, p * NJ_C + j, 0)),
        out_shape=jax.ShapeDtypeStruct((NCH, H, W), jnp.float32),
        compiler_params=pltpu.CompilerParams(
            dimension_semantics=("parallel", "arbitrary", "arbitrary")),
    )(image, lut)

    return out


# speedup vs baseline: 1.2660x; 1.2660x over previous
import jax
import jax.numpy as jnp
from jax.experimental import pallas as pl
from jax.experimental.pallas import tpu as pltpu

H, W = 2048, 4096
NLAB = 3
NCH = 3 + NLAB
HH = H // 2
BINS = 256

BR_A = 128
NJ_A = HH // BR_A
TPB = (BR_A // 16) * (W // 128)
BR_C = 128
NJ_C = HH // BR_C


NBB = 16
NPASS = BINS // NBB
NCOLS = W // 128


def _hist_kernel(img_ref, hist_ref, vals):
    j = pl.program_id(2)
    one = jnp.ones((), jnp.bfloat16)
    zero = jnp.zeros((), jnp.bfloat16)

    @pl.when(j == 0)
    def _():
        hist_ref[0, 0] = jnp.zeros((BINS, 128), jnp.float32)

    vals[...] = jnp.clip(jnp.floor(img_ref[0]), 0.0, 255.0).astype(jnp.bfloat16)

    for bb in range(NPASS):
        def col_body(i, accs):
            cs = pl.multiple_of(i * 128, 128)
            out = list(accs)
            for s in range(BR_A // 16):
                tl = vals[s * 16:(s + 1) * 16, pl.ds(cs, 128)]
                for k in range(NBB):
                    out[k] = out[k] + jnp.where(
                        tl == jnp.bfloat16(bb * NBB + k), one, zero)
            return tuple(out)

        init = tuple(jnp.zeros((16, 128), jnp.bfloat16) for _ in range(NBB))
        accs = jax.lax.fori_loop(0, NCOLS, col_body, init)

        for k in range(NBB):
            b = bb * NBB + k
            r = jnp.sum(accs[k].astype(jnp.float32), axis=0, keepdims=True)
            hist_ref[0, 0, b:b + 1, :] = hist_ref[0, 0, b:b + 1, :] + r


def _fdiv(a, d):
    q = jnp.floor(a / d)
    r = a - q * d
    for _ in range(3):
        over = r >= d
        q = q + jnp.where(over, 1.0, 0.0)
        r = r - jnp.where(over, d, 0.0)
        under = r < 0.0
        q = q - jnp.where(under, 1.0, 0.0)
        r = r + jnp.where(under, d, 0.0)
    return q


def _shift_rows_down(x, k):
    return jnp.pad(x, ((k, 0), (0, 0)))[:BINS, :]


def _lut_kernel(hist_ref, lut_ref):
    idx = jax.lax.broadcasted_iota(jnp.int32, (BINS, 1), 0).astype(jnp.float32)

    for c in range(3):
        hl = hist_ref[0, c] + hist_ref[1, c]
        h = jnp.sum(hl, axis=-1, keepdims=True)
        cum = h
        for k in (1, 2, 4, 8, 16, 32, 64, 128):
            cum = cum + _shift_rows_down(cum, k)
        total = jnp.sum(h, keepdims=True)
        masked = jnp.where(h > 0.0, idx, -1.0)
        last_nz = jnp.max(masked, keepdims=True)
        h_last = jnp.sum(jnp.where(idx == last_nz, h, 0.0), keepdims=True)
        step = _fdiv(total - h_last, jnp.full((1, 1), 255.0))
        d = jnp.maximum(step, 1.0)
        a = cum + jnp.floor(step * 0.5)
        q = _fdiv(a, d)
        lut = jnp.clip(_shift_rows_down(q, 1), 0.0, 255.0)
        lut = jnp.where(step == 0.0, idx, lut)
        lut_ref[c] = lut


def _remap_kernel(img_ref, lut_ref, out_ref):
    c = pl.program_id(1)

    @pl.when(c < 3)
    def _():
        v = img_ref[0]
        idx = jnp.clip(v, 0.0, 255.0).astype(jnp.int32)
        idxm = jnp.bitwise_and(idx, 127)
        lo_rows = jnp.broadcast_to(lut_ref[0, 0:1, :], (BR_C, 128))
        hi_rows = jnp.broadcast_to(lut_ref[0, 1:2, :], (BR_C, 128))
        g_lo = jnp.take_along_axis(lo_rows, idxm, axis=1)
        g_hi = jnp.take_along_axis(hi_rows, idxm, axis=1)
        out_ref[0] = jnp.where(idx >= 128, g_hi, g_lo)

    @pl.when(c >= 3)
    def _():
        out_ref[0] = img_ref[0]


def kernel(image):
    part = pl.pallas_call(
        _hist_kernel,
        grid=(2, 3, NJ_A),
        in_specs=[pl.BlockSpec((1, BR_A, W),
                               lambda p, c, j: (c, p * NJ_A + j, 0))],
        out_specs=pl.BlockSpec((1, 1, BINS, 128), lambda p, c, j: (p, c, 0, 0)),
        out_shape=jax.ShapeDtypeStruct((2, 3, BINS, 128), jnp.float32),
        scratch_shapes=[
            pltpu.VMEM((BR_A, W), jnp.bfloat16),
        ],
        compiler_params=pltpu.CompilerParams(
            dimension_semantics=("parallel", "arbitrary", "arbitrary")),
    )(image)

    lut3d = pl.pallas_call(
        _lut_kernel,
        grid=(1,),
        in_specs=[pl.BlockSpec((2, 3, BINS, 128), lambda i: (0, 0, 0, 0))],
        out_specs=pl.BlockSpec((3, BINS, 1), lambda i: (0, 0, 0)),
        out_shape=jax.ShapeDtypeStruct((3, BINS, 1), jnp.float32),
        compiler_params=pltpu.CompilerParams(
            dimension_semantics=("arbitrary",)),
    )(part)

    lut = lut3d.reshape(3, 2, 128)

    out = pl.pallas_call(
        _remap_kernel,
        grid=(2, NCH, NJ_C),
        in_specs=[
            pl.BlockSpec((1, BR_C, W), lambda p, c, j: (c, p * NJ_C + j, 0)),
            pl.BlockSpec((1, 2, 128),
                         lambda p, c, j: (jnp.minimum(c, 2), 0, 0)),
        ],
        out_specs=pl.BlockSpec((1, BR_C, W),
                               lambda p, c, j: (c, p * NJ_C + j, 0)),
        out_shape=jax.ShapeDtypeStruct((NCH, H, W), jnp.float32),
        compiler_params=pltpu.CompilerParams(
            dimension_semantics=("parallel", "arbitrary", "arbitrary")),
    )(image, lut)

    return out


# speedup vs baseline: 1.3481x; 1.0649x over previous
import jax
import jax.numpy as jnp
from jax.experimental import pallas as pl
from jax.experimental.pallas import tpu as pltpu

H, W = 2048, 4096
NLAB = 3
NCH = 3 + NLAB
HH = H // 2
BINS = 256

BR_A = 128
NJ_A = HH // BR_A
TPB = (BR_A // 16) * (W // 128)
BR_C = 128
NJ_C = HH // BR_C


NBB = 16
NPASS = BINS // NBB
NCOLS = W // 128


def _hist_kernel(img_ref, hist_ref, vals):
    j = pl.program_id(2)
    one = jnp.ones((), jnp.bfloat16)
    zero = jnp.zeros((), jnp.bfloat16)

    @pl.when(j == 0)
    def _():
        hist_ref[0, 0] = jnp.zeros((BINS, 128), jnp.float32)

    vals[...] = jnp.clip(jnp.floor(img_ref[0]), 0.0, 255.0).astype(jnp.bfloat16)

    UNR = 4
    for bb in range(NPASS):
        def col_body(i, accs):
            out = list(accs)
            for u in range(UNR):
                cs = pl.multiple_of((i * UNR + u) * 128, 128)
                for s in range(BR_A // 16):
                    tl = vals[s * 16:(s + 1) * 16, pl.ds(cs, 128)]
                    for k in range(NBB):
                        out[k] = out[k] + jnp.where(
                            tl == jnp.bfloat16(bb * NBB + k), one, zero)
            return tuple(out)

        init = tuple(jnp.zeros((16, 128), jnp.bfloat16) for _ in range(NBB))
        accs = jax.lax.fori_loop(0, NCOLS // UNR, col_body, init)

        for k in range(NBB):
            b = bb * NBB + k
            r = jnp.sum(accs[k].astype(jnp.float32), axis=0, keepdims=True)
            hist_ref[0, 0, b:b + 1, :] = hist_ref[0, 0, b:b + 1, :] + r


def _fdiv(a, d):
    q = jnp.floor(a / d)
    r = a - q * d
    for _ in range(3):
        over = r >= d
        q = q + jnp.where(over, 1.0, 0.0)
        r = r - jnp.where(over, d, 0.0)
        under = r < 0.0
        q = q - jnp.where(under, 1.0, 0.0)
        r = r + jnp.where(under, d, 0.0)
    return q


def _shift_rows_down(x, k):
    return jnp.pad(x, ((k, 0), (0, 0)))[:BINS, :]


def _lut_kernel(hist_ref, lut_ref):
    idx = jax.lax.broadcasted_iota(jnp.int32, (BINS, 1), 0).astype(jnp.float32)

    for c in range(3):
        hl = hist_ref[0, c] + hist_ref[1, c]
        h = jnp.sum(hl, axis=-1, keepdims=True)
        cum = h
        for k in (1, 2, 4, 8, 16, 32, 64, 128):
            cum = cum + _shift_rows_down(cum, k)
        total = jnp.sum(h, keepdims=True)
        masked = jnp.where(h > 0.0, idx, -1.0)
        last_nz = jnp.max(masked, keepdims=True)
        h_last = jnp.sum(jnp.where(idx == last_nz, h, 0.0), keepdims=True)
        step = _fdiv(total - h_last, jnp.full((1, 1), 255.0))
        d = jnp.maximum(step, 1.0)
        a = cum + jnp.floor(step * 0.5)
        q = _fdiv(a, d)
        lut = jnp.clip(_shift_rows_down(q, 1), 0.0, 255.0)
        lut = jnp.where(step == 0.0, idx, lut)
        lut_ref[c] = lut


def _remap_kernel(img_ref, lut_ref, out_ref):
    c = pl.program_id(1)

    @pl.when(c < 3)
    def _():
        v = img_ref[0]
        idx = jnp.clip(v, 0.0, 255.0).astype(jnp.int32)
        idxm = jnp.bitwise_and(idx, 127)
        lo_rows = jnp.broadcast_to(lut_ref[0, 0:1, :], (BR_C, 128))
        hi_rows = jnp.broadcast_to(lut_ref[0, 1:2, :], (BR_C, 128))
        g_lo = jnp.take_along_axis(lo_rows, idxm, axis=1)
        g_hi = jnp.take_along_axis(hi_rows, idxm, axis=1)
        out_ref[0] = jnp.where(idx >= 128, g_hi, g_lo)

    @pl.when(c >= 3)
    def _():
        out_ref[0] = img_ref[0]


def kernel(image):
    part = pl.pallas_call(
        _hist_kernel,
        grid=(2, 3, NJ_A),
        in_specs=[pl.BlockSpec((1, BR_A, W),
                               lambda p, c, j: (c, p * NJ_A + j, 0))],
        out_specs=pl.BlockSpec((1, 1, BINS, 128), lambda p, c, j: (p, c, 0, 0)),
        out_shape=jax.ShapeDtypeStruct((2, 3, BINS, 128), jnp.float32),
        scratch_shapes=[
            pltpu.VMEM((BR_A, W), jnp.bfloat16),
        ],
        compiler_params=pltpu.CompilerParams(
            dimension_semantics=("arbitrary", "arbitrary", "arbitrary")),
    )(image)

    lut3d = pl.pallas_call(
        _lut_kernel,
        grid=(1,),
        in_specs=[pl.BlockSpec((2, 3, BINS, 128), lambda i: (0, 0, 0, 0))],
        out_specs=pl.BlockSpec((3, BINS, 1), lambda i: (0, 0, 0)),
        out_shape=jax.ShapeDtypeStruct((3, BINS, 1), jnp.float32),
        compiler_params=pltpu.CompilerParams(
            dimension_semantics=("arbitrary",)),
    )(part)

    lut = lut3d.reshape(3, 2, 128)

    out = pl.pallas_call(
        _remap_kernel,
        grid=(2, NCH, NJ_C),
        in_specs=[
            pl.BlockSpec((1, BR_C, W), lambda p, c, j: (c, p * NJ_C + j, 0)),
            pl.BlockSpec((1, 2, 128),
                         lambda p, c, j: (jnp.minimum(c, 2), 0, 0)),
        ],
        out_specs=pl.BlockSpec((1, BR_C, W),
                               lambda p, c, j: (c, p * NJ_C + j, 0)),
        out_shape=jax.ShapeDtypeStruct((NCH, H, W), jnp.float32),
        compiler_params=pltpu.CompilerParams(
            dimension_semantics=("arbitrary", "arbitrary", "arbitrary")),
    )(image, lut)

    return out
